# Initial kernel scaffold; baseline (speedup 1.0000x reference)
import jax
import jax.numpy as jnp
from jax.experimental import pallas as pl


def kernel(queries, keys, values, attn_mask):
    raise NotImplementedError("write your pallas kernel here")


# speedup vs baseline: 3.2084x; 3.2084x over previous
import functools

import numpy as np
import jax
import jax.numpy as jnp
from jax import lax
from jax.experimental import pallas as pl
from jax.experimental.pallas import tpu as pltpu
from jax.experimental.pallas import tpu_sc as plsc

_L, _H, _D, _U = 4096, 12, 64, 45
_QB = 512
_PREC = lax.Precision.HIGHEST

_idx_np = np.asarray(jax.random.randint(jax.random.key(42), (_L, _U), 0, _L))
_counts_np = np.zeros((_L, _L), np.uint8)
np.add.at(_counts_np, (np.arange(_L)[:, None], _idx_np), 1)
del _idx_np


def _m_body(q_ref, k_ref, c_ref, m_ref):
    qb = pl.program_id(1)
    s = lax.dot_general(q_ref[0], k_ref[0], (((1,), (1,)), ((), ())),
                        preferred_element_type=jnp.float32,
                        precision=_PREC)
    cf = c_ref[pl.ds(qb * _QB, _QB), :].astype(jnp.float32)
    maxp = jnp.max(jnp.where(cf > 0.0, s, -jnp.inf), axis=1)
    sump = jnp.sum(s * cf, axis=1)
    m_ref[0, 0, :] = maxp - sump * (1.0 / _L)


_m_call = pl.pallas_call(
    _m_body,
    grid=(_H, _L // _QB),
    in_specs=[
        pl.BlockSpec((1, _QB, _D), lambda h, qb: (h, qb, 0)),
        pl.BlockSpec((1, _L, _D), lambda h, qb: (h, 0, 0)),
        pl.BlockSpec((_L, _L), lambda h, qb: (0, 0)),
    ],
    out_specs=pl.BlockSpec((1, 1, _QB), lambda h, qb: (h * (_L // _QB) + qb, 0, 0)),
    out_shape=jax.ShapeDtypeStruct((_H * (_L // _QB), 1, _QB), jnp.float32),
)


def _sel_body(m_ref, q_ref, k_ref, v_ref, a_ref, gi_ref, mb_ref, ms_ref, qr_ref):
    h = pl.program_id(0)
    ms_ref[...] = m_ref[pl.ds(h, 1), :]
    qr_ref[...] = jnp.zeros((48, _D), jnp.float32)
    ii = lax.broadcasted_iota(jnp.int32, (1, _L), 1)
    i48 = lax.broadcasted_iota(jnp.int32, (1, 48), 1)

    def body(i, gvec):
        row = ms_ref[...]
        mx = jnp.max(row)
        idx = jnp.min(jnp.where(row == mx, ii, jnp.int32(_L)))
        qr_ref[pl.ds(i, 1), :] = q_ref[0, pl.ds(idx, 1), :]
        ms_ref[...] = jnp.where(ii == idx, -jnp.inf, row)
        return jnp.where(i48 == i, idx * _H + h, gvec)

    gvec = lax.fori_loop(0, _U, body, jnp.full((1, 48), _L * _H, jnp.int32))
    gi_ref[...] = gvec.reshape(1, 1, 48)
    scores = lax.dot_general(qr_ref[...], k_ref[0], (((1,), (1,)), ((), ())),
                             preferred_element_type=jnp.float32,
                             precision=_PREC) * 0.125
    smax = jnp.max(scores, axis=1, keepdims=True)
    p = jnp.exp(scores - smax)
    attn = p / jnp.sum(p, axis=1, keepdims=True)
    a_ref[0] = lax.dot_general(attn, v_ref[0], (((1,), (0,)), ((), ())),
                               preferred_element_type=jnp.float32,
                               precision=_PREC)
    vmean = jnp.sum(v_ref[0], axis=0, keepdims=True) * (1.0 / _L)
    mb_ref[0, 0] = jnp.broadcast_to(vmean, (8, _D))


_sel_call = pl.pallas_call(
    _sel_body,
    grid=(_H,),
    in_specs=[
        pl.BlockSpec((_H, _L), lambda h: (0, 0)),
        pl.BlockSpec((1, _L, _D), lambda h: (h, 0, 0)),
        pl.BlockSpec((1, _L, _D), lambda h: (h, 0, 0)),
        pl.BlockSpec((1, _L, _D), lambda h: (h, 0, 0)),
    ],
    out_specs=[
        pl.BlockSpec((1, 48, _D), lambda h: (h, 0, 0)),
        pl.BlockSpec((1, 1, 48), lambda h: (h, 0, 0)),
        pl.BlockSpec((1, 1, 8, _D), lambda h: (h, 0, 0, 0)),
    ],
    out_shape=[
        jax.ShapeDtypeStruct((_H, 48, _D), jnp.float32),
        jax.ShapeDtypeStruct((_H, 1, 48), jnp.int32),
        jax.ShapeDtypeStruct((_H, 1, 8, _D), jnp.float32),
    ],
    scratch_shapes=[
        pltpu.VMEM((1, _L), jnp.float32),
        pltpu.VMEM((48, _D), jnp.float32),
    ],
)

_ROWS = _L * _H
_FILL = 1536


@functools.cache
def _build_assemble():
    @functools.partial(
        pl.kernel,
        out_type=jax.ShapeDtypeStruct((_ROWS + 8, _D), jnp.float32),
        mesh=plsc.VectorSubcoreMesh(core_axis_name="c", subcore_axis_name="s"),
        compiler_params=pltpu.CompilerParams(use_tc_tiling_on_sc=False),
        scratch_types=[
            pltpu.VMEM((96, _D), jnp.float32),
            pltpu.VMEM((1024,), jnp.int32),
            pltpu.VMEM((1024,), jnp.int32),
            pltpu.VMEM((1024, _D), jnp.float32),
            pltpu.SemaphoreType.DMA,
            pltpu.SemaphoreType.DMA,
        ],
    )
    def _assemble(mp_hbm, a_hbm, gi_hbm, out_hbm,
                  blk_v, idxa_v, idxb_v, rows_v, sem_f, sem_s):
        w = lax.axis_index("s") * 2 + lax.axis_index("c")
        lo = w * _FILL
        pltpu.sync_copy(mp_hbm, blk_v)
        pltpu.sync_copy(gi_hbm, idxa_v)
        pltpu.sync_copy(a_hbm, rows_v)
        for j in range(1024 // 16):
            x = idxa_v[pl.ds(j * 16, 16)]
            ok = (x >= lo) & (x < lo + _FILL)
            idxb_v[pl.ds(j * 16, 16)] = jnp.where(ok, x, _ROWS)
        copies = [
            pltpu.async_copy(blk_v, out_hbm.at[pl.ds(w * _FILL + i * 96, 96)], sem_f)
            for i in range(_FILL // 96)
        ]
        for cp in copies:
            cp.wait()
        pltpu.async_copy(rows_v, out_hbm.at[idxb_v], sem_s).wait()

    return _assemble


def kernel(queries, keys, values, attn_mask):
    q2 = queries[0].transpose(1, 0, 2)
    k2 = keys[0].transpose(1, 0, 2)
    v2 = values[0].transpose(1, 0, 2)
    counts = jnp.asarray(_counts_np)
    m2 = _m_call(q2, k2, counts).reshape(_H, _L)
    a48, gi3, mb4 = _sel_call(m2, q2, k2, v2)
    meanpat = mb4.reshape(_H, 8, _D).transpose(1, 0, 2).reshape(96, _D)
    gidx = gi3.reshape(_H, 48)[:, :_U].reshape(-1)
    a540 = a48[:, :_U, :].reshape(-1, _D)
    gpad = jnp.full((1024,), _ROWS, jnp.int32).at[:540].set(gidx)
    apad = jnp.zeros((1024, _D), jnp.float32).at[:540].set(a540)
    out = _build_assemble()(meanpat, apad, gpad)
    return out[:_ROWS].reshape(1, _L, _H, _D)


# speedup vs baseline: 4.8526x; 1.5125x over previous
import functools

import numpy as np
import jax
import jax.numpy as jnp
from jax import lax
from jax.experimental import pallas as pl
from jax.experimental.pallas import tpu as pltpu
from jax.experimental.pallas import tpu_sc as plsc

_L, _H, _D, _U = 4096, 12, 64, 45
_QB = 512
_PREC = lax.Precision.HIGHEST

_idx_np = np.asarray(jax.random.randint(jax.random.key(42), (_L, _U), 0, _L))
_counts_np = np.zeros((_L, _L), np.uint8)
np.add.at(_counts_np, (np.arange(_L)[:, None], _idx_np), 1)
del _idx_np


def _m_body(q_ref, k_ref, c_ref, m_ref):
    qb = pl.program_id(1)
    s = lax.dot_general(q_ref[0], k_ref[0], (((1,), (1,)), ((), ())),
                        preferred_element_type=jnp.float32,
                        precision=_PREC)
    cf = c_ref[pl.ds(qb * _QB, _QB), :].astype(jnp.float32)
    maxp = jnp.max(jnp.where(cf > 0.0, s, -jnp.inf), axis=1)
    sump = jnp.sum(s * cf, axis=1)
    m_ref[0, 0, :] = maxp - sump * (1.0 / _L)


_m_call = pl.pallas_call(
    _m_body,
    grid=(_H, _L // _QB),
    in_specs=[
        pl.BlockSpec((1, _QB, _D), lambda h, qb: (h, qb, 0)),
        pl.BlockSpec((1, _L, _D), lambda h, qb: (h, 0, 0)),
        pl.BlockSpec((_L, _L), lambda h, qb: (0, 0)),
    ],
    out_specs=pl.BlockSpec((1, 1, _QB), lambda h, qb: (h * (_L // _QB) + qb, 0, 0)),
    out_shape=jax.ShapeDtypeStruct((_H * (_L // _QB), 1, _QB), jnp.float32),
)


def _sel_body(m_ref, q_ref, k_ref, v_ref, a_ref, gi_ref, mb_ref, ms_ref, qr_ref):
    h = pl.program_id(0)
    ms_ref[...] = m_ref[pl.ds(h, 1), :]
    qr_ref[...] = jnp.zeros((48, _D), jnp.float32)
    ii = lax.broadcasted_iota(jnp.int32, (1, _L), 1)
    i48 = lax.broadcasted_iota(jnp.int32, (1, 48), 1)

    def body(i, gvec):
        row = ms_ref[...]
        mx = jnp.max(row)
        idx = jnp.min(jnp.where(row == mx, ii, jnp.int32(_L)))
        qr_ref[pl.ds(i, 1), :] = q_ref[0, pl.ds(idx, 1), :]
        ms_ref[...] = jnp.where(ii == idx, -jnp.inf, row)
        return jnp.where(i48 == i, idx * _H + h, gvec)

    gvec = lax.fori_loop(0, _U, body, jnp.full((1, 48), _L * _H, jnp.int32))
    gi_ref[...] = gvec.reshape(1, 1, 48)
    scores = lax.dot_general(qr_ref[...], k_ref[0], (((1,), (1,)), ((), ())),
                             preferred_element_type=jnp.float32,
                             precision=_PREC) * 0.125
    smax = jnp.max(scores, axis=1, keepdims=True)
    p = jnp.exp(scores - smax)
    attn = p / jnp.sum(p, axis=1, keepdims=True)
    a_ref[0] = lax.dot_general(attn, v_ref[0], (((1,), (0,)), ((), ())),
                               preferred_element_type=jnp.float32,
                               precision=_PREC)
    vmean = jnp.sum(v_ref[0], axis=0, keepdims=True) * (1.0 / _L)
    mb_ref[0, 0] = jnp.broadcast_to(vmean, (8, _D))


_sel_call = pl.pallas_call(
    _sel_body,
    grid=(_H,),
    in_specs=[
        pl.BlockSpec((_H, _L), lambda h: (0, 0)),
        pl.BlockSpec((1, _L, _D), lambda h: (h, 0, 0)),
        pl.BlockSpec((1, _L, _D), lambda h: (h, 0, 0)),
        pl.BlockSpec((1, _L, _D), lambda h: (h, 0, 0)),
    ],
    out_specs=[
        pl.BlockSpec((1, 48, _D), lambda h: (h, 0, 0)),
        pl.BlockSpec((1, 1, 48), lambda h: (h, 0, 0)),
        pl.BlockSpec((1, 1, 8, _D), lambda h: (h, 0, 0, 0)),
    ],
    out_shape=[
        jax.ShapeDtypeStruct((_H, 48, _D), jnp.float32),
        jax.ShapeDtypeStruct((_H, 1, 48), jnp.int32),
        jax.ShapeDtypeStruct((_H, 1, 8, _D), jnp.float32),
    ],
    scratch_shapes=[
        pltpu.VMEM((1, _L), jnp.float32),
        pltpu.VMEM((48, _D), jnp.float32),
    ],
)

_ROWS = _L * _H
_FILL = 1536
_TRASH = 1024


@functools.cache
def _build_assemble():
    @functools.partial(
        pl.kernel,
        out_type=jax.ShapeDtypeStruct((_ROWS + _TRASH, _D), jnp.float32),
        mesh=plsc.VectorSubcoreMesh(core_axis_name="c", subcore_axis_name="s"),
        compiler_params=pltpu.CompilerParams(use_tc_tiling_on_sc=False),
        scratch_types=[
            pltpu.VMEM((96, _D), jnp.float32),
            pltpu.VMEM((1024,), jnp.int32),
            pltpu.VMEM((1024,), jnp.int32),
            pltpu.VMEM((1024, _D), jnp.float32),
            pltpu.SemaphoreType.DMA,
            pltpu.SemaphoreType.DMA,
        ],
    )
    def _assemble(mp_hbm, a_hbm, gi_hbm, out_hbm,
                  blk_v, idxa_v, idxb_v, rows_v, sem_f, sem_s):
        w = lax.axis_index("s") * 2 + lax.axis_index("c")
        lo = w * _FILL
        pltpu.sync_copy(mp_hbm, blk_v)
        pltpu.sync_copy(gi_hbm, idxa_v)
        pltpu.sync_copy(a_hbm, rows_v)
        lane = lax.iota(jnp.int32, 16)
        for j in range(1024 // 16):
            x = idxa_v[pl.ds(j * 16, 16)]
            ok = (x >= lo) & (x < lo + _FILL)
            idxb_v[pl.ds(j * 16, 16)] = jnp.where(ok, x, _ROWS + j * 16 + lane)
        copies = [
            pltpu.async_copy(blk_v, out_hbm.at[pl.ds(w * _FILL + i * 96, 96)], sem_f)
            for i in range(_FILL // 96)
        ]
        for cp in copies:
            cp.wait()
        pltpu.async_copy(rows_v, out_hbm.at[idxb_v], sem_s).wait()

    return _assemble


def kernel(queries, keys, values, attn_mask):
    q2 = queries[0].transpose(1, 0, 2)
    k2 = keys[0].transpose(1, 0, 2)
    v2 = values[0].transpose(1, 0, 2)
    counts = jnp.asarray(_counts_np)
    m2 = _m_call(q2, k2, counts).reshape(_H, _L)
    a48, gi3, mb4 = _sel_call(m2, q2, k2, v2)
    meanpat = mb4.reshape(_H, 8, _D).transpose(1, 0, 2).reshape(96, _D)
    gidx = gi3.reshape(_H, 48)[:, :_U].reshape(-1)
    a540 = a48[:, :_U, :].reshape(-1, _D)
    gpad = jnp.full((1024,), _ROWS, jnp.int32).at[:540].set(gidx)
    apad = jnp.zeros((1024, _D), jnp.float32).at[:540].set(a540)
    out = _build_assemble()(meanpat, apad, gpad)
    return out[:_ROWS].reshape(1, _L, _H, _D)


# speedup vs baseline: 4.8665x; 1.0029x over previous
import functools

import numpy as np
import jax
import jax.numpy as jnp
from jax import lax
from jax.experimental import pallas as pl
from jax.experimental.pallas import tpu as pltpu
from jax.experimental.pallas import tpu_sc as plsc

_L, _H, _D, _U = 4096, 12, 64, 45
_QB = 512
_PREC = lax.Precision.HIGHEST


def _threefry2x32(k0, k1, x0, x1):
    def rotl(x, d):
        return ((x << np.uint32(d)) | (x >> np.uint32(32 - d))).astype(np.uint32)

    rots = ((13, 15, 26, 6), (17, 29, 16, 24))
    ks = (np.uint32(k0), np.uint32(k1),
          np.uint32(k0) ^ np.uint32(k1) ^ np.uint32(0x1BD11BDA))
    x0 = (x0 + ks[0]).astype(np.uint32)
    x1 = (x1 + ks[1]).astype(np.uint32)
    for i in range(5):
        for r in rots[i % 2]:
            x0 = (x0 + x1).astype(np.uint32)
            x1 = rotl(x1, r) ^ x0
        x0 = (x0 + ks[(i + 1) % 3]).astype(np.uint32)
        x1 = (x1 + ks[(i + 2) % 3] + np.uint32(i + 1)).astype(np.uint32)
    return x0, x1


def _sample_indices(seed, L, U):
    b1, b2 = _threefry2x32(np.uint32(0), np.uint32(seed),
                           np.zeros(2, np.uint32), np.arange(2, dtype=np.uint32))
    n = L * U
    c1, c2 = _threefry2x32(b1[1], b2[1],
                           np.zeros(n, np.uint32), np.arange(n, dtype=np.uint32))
    return ((c1 ^ c2) % np.uint32(L)).astype(np.int32).reshape(L, U)


_idx_np = _sample_indices(42, _L, _U)
_counts_np = np.zeros((_L, _L), np.uint8)
np.add.at(_counts_np, (np.arange(_L)[:, None], _idx_np), 1)
del _idx_np


def _m_body(q_ref, k_ref, c_ref, m_ref):
    qb = pl.program_id(1)
    q = q_ref[0]
    maxp = None
    sump = None
    kc = 1024
    for t in range(_L // kc):
        s = lax.dot_general(q, k_ref[0, pl.ds(t * kc, kc), :],
                            (((1,), (1,)), ((), ())),
                            preferred_element_type=jnp.float32,
                            precision=_PREC)
        cf = c_ref[pl.ds(qb * _QB, _QB),
                   pl.ds(t * kc, kc)].astype(jnp.float32)
        mx = jnp.max(jnp.where(cf > 0.0, s, -jnp.inf), axis=1)
        sm = jnp.sum(s * cf, axis=1)
        maxp = mx if maxp is None else jnp.maximum(maxp, mx)
        sump = sm if sump is None else sump + sm
    m_ref[0, 0, :] = maxp - sump * (1.0 / _L)


_m_call = pl.pallas_call(
    _m_body,
    grid=(_H, _L // _QB),
    in_specs=[
        pl.BlockSpec((1, _QB, _D), lambda h, qb: (h, qb, 0)),
        pl.BlockSpec((1, _L, _D), lambda h, qb: (h, 0, 0)),
        pl.BlockSpec((_L, _L), lambda h, qb: (0, 0)),
    ],
    out_specs=pl.BlockSpec((1, 1, _QB), lambda h, qb: (h * (_L // _QB) + qb, 0, 0)),
    out_shape=jax.ShapeDtypeStruct((_H * (_L // _QB), 1, _QB), jnp.float32),
)


def _sel_body(m_ref, q_ref, k_ref, v_ref, a_ref, gi_ref, mb_ref, ms_ref, qr_ref):
    h = pl.program_id(0)
    ms_ref[...] = m_ref[pl.ds(h, 1), :]
    qr_ref[...] = jnp.zeros((48, _D), jnp.float32)
    ii = lax.broadcasted_iota(jnp.int32, (1, _L), 1)
    i48 = lax.broadcasted_iota(jnp.int32, (1, 48), 1)

    def body(i, gvec):
        row = ms_ref[...]
        mx = jnp.max(row)
        idx = jnp.min(jnp.where(row == mx, ii, jnp.int32(_L)))
        qr_ref[pl.ds(i, 1), :] = q_ref[0, pl.ds(idx, 1), :]
        ms_ref[...] = jnp.where(ii == idx, -jnp.inf, row)
        return jnp.where(i48 == i, idx * _H + h, gvec)

    gvec = lax.fori_loop(0, _U, body, jnp.full((1, 48), _L * _H, jnp.int32))
    gi_ref[...] = gvec.reshape(1, 1, 48)
    scores = lax.dot_general(qr_ref[...], k_ref[0], (((1,), (1,)), ((), ())),
                             preferred_element_type=jnp.float32,
                             precision=_PREC) * 0.125
    smax = jnp.max(scores, axis=1, keepdims=True)
    p = jnp.exp(scores - smax)
    attn = p / jnp.sum(p, axis=1, keepdims=True)
    a_ref[0] = lax.dot_general(attn, v_ref[0], (((1,), (0,)), ((), ())),
                               preferred_element_type=jnp.float32,
                               precision=_PREC)
    vmean = jnp.sum(v_ref[0], axis=0, keepdims=True) * (1.0 / _L)
    mb_ref[0, 0] = jnp.broadcast_to(vmean, (8, _D))


_sel_call = pl.pallas_call(
    _sel_body,
    grid=(_H,),
    in_specs=[
        pl.BlockSpec((_H, _L), lambda h: (0, 0)),
        pl.BlockSpec((1, _L, _D), lambda h: (h, 0, 0)),
        pl.BlockSpec((1, _L, _D), lambda h: (h, 0, 0)),
        pl.BlockSpec((1, _L, _D), lambda h: (h, 0, 0)),
    ],
    out_specs=[
        pl.BlockSpec((1, 48, _D), lambda h: (h, 0, 0)),
        pl.BlockSpec((1, 1, 48), lambda h: (h, 0, 0)),
        pl.BlockSpec((1, 1, 8, _D), lambda h: (h, 0, 0, 0)),
    ],
    out_shape=[
        jax.ShapeDtypeStruct((_H, 48, _D), jnp.float32),
        jax.ShapeDtypeStruct((_H, 1, 48), jnp.int32),
        jax.ShapeDtypeStruct((_H, 1, 8, _D), jnp.float32),
    ],
    scratch_shapes=[
        pltpu.VMEM((1, _L), jnp.float32),
        pltpu.VMEM((48, _D), jnp.float32),
    ],
)

_ROWS = _L * _H
_FILL = 1536
_TRASH = 1024


@functools.cache
def _build_assemble():
    @functools.partial(
        pl.kernel,
        out_type=jax.ShapeDtypeStruct((_ROWS + _TRASH, _D), jnp.float32),
        mesh=plsc.VectorSubcoreMesh(core_axis_name="c", subcore_axis_name="s"),
        compiler_params=pltpu.CompilerParams(use_tc_tiling_on_sc=False),
        scratch_types=[
            pltpu.VMEM((96, _D), jnp.float32),
            pltpu.VMEM((1024,), jnp.int32),
            pltpu.VMEM((1024,), jnp.int32),
            pltpu.VMEM((1024, _D), jnp.float32),
            pltpu.SemaphoreType.DMA,
            pltpu.SemaphoreType.DMA,
        ],
    )
    def _assemble(mp_hbm, a_hbm, gi_hbm, out_hbm,
                  blk_v, idxa_v, idxb_v, rows_v, sem_f, sem_s):
        w = lax.axis_index("s") * 2 + lax.axis_index("c")
        lo = w * _FILL
        pltpu.sync_copy(mp_hbm, blk_v)
        pltpu.sync_copy(gi_hbm, idxa_v)
        pltpu.sync_copy(a_hbm, rows_v)
        lane = lax.iota(jnp.int32, 16)
        for j in range(1024 // 16):
            x = idxa_v[pl.ds(j * 16, 16)]
            ok = (x >= lo) & (x < lo + _FILL)
            idxb_v[pl.ds(j * 16, 16)] = jnp.where(ok, x, _ROWS + j * 16 + lane)
        copies = [
            pltpu.async_copy(blk_v, out_hbm.at[pl.ds(w * _FILL + i * 96, 96)], sem_f)
            for i in range(_FILL // 96)
        ]
        for cp in copies:
            cp.wait()
        pltpu.async_copy(rows_v, out_hbm.at[idxb_v], sem_s).wait()

    return _assemble


def kernel(queries, keys, values, attn_mask):
    q2 = queries[0].transpose(1, 0, 2)
    k2 = keys[0].transpose(1, 0, 2)
    v2 = values[0].transpose(1, 0, 2)
    counts = jnp.asarray(_counts_np)
    m2 = _m_call(q2, k2, counts).reshape(_H, _L)
    a48, gi3, mb4 = _sel_call(m2, q2, k2, v2)
    meanpat = mb4.reshape(_H, 8, _D).transpose(1, 0, 2).reshape(96, _D)
    gidx = gi3.reshape(_H, 48)[:, :_U].reshape(-1)
    a540 = a48[:, :_U, :].reshape(-1, _D)
    gpad = jnp.full((1024,), _ROWS, jnp.int32).at[:540].set(gidx)
    apad = jnp.zeros((1024, _D), jnp.float32).at[:540].set(a540)
    out = _build_assemble()(meanpat, apad, gpad)
    return out[:_ROWS].reshape(1, _L, _H, _D)
